# Initial kernel scaffold; baseline (speedup 1.0000x reference)
import jax
import jax.numpy as jnp
from jax.experimental import pallas as pl


def kernel(context_words, center_word, negative_samples, in_embeddings, out_embeddings):
    raise NotImplementedError("write your pallas kernel here")


# speedup vs baseline: 5.2452x; 5.2452x over previous
import jax
import jax.numpy as jnp
from jax import lax
from jax.experimental import pallas as pl
from jax.experimental.pallas import tpu as pltpu
from jax.experimental.pallas import tpu_sc as plsc

_DIM = 64
_BATCH = 16384
_CTX = 10
_NEG = 20
_NW = 32
_BPW = _BATCH // _NW
_CB = 32
_NCH = _BPW // _CB
_SCORE_COLS = 32
_L = 16


def _sc_scores_body(ctx_idx_hbm, cen_idx_hbm, neg_idx_hbm, in_emb_hbm,
                    out_emb_hbm, scores_hbm,
                    idx_ctx, idx_cen, idx_neg,
                    rows_ctx, rows_cen, rows_neg, scores_v, sem):
    nc = plsc.get_sparse_core_info().num_cores
    wid = lax.axis_index("s") * nc + lax.axis_index("c")
    tile_base = wid * _BPW

    def chunk_body(ch, carry):
        cbase = tile_base + ch * _CB
        pltpu.sync_copy(ctx_idx_hbm.at[pl.ds(cbase * _CTX, _CB * _CTX)],
                        idx_ctx)
        pltpu.sync_copy(cen_idx_hbm.at[pl.ds(cbase, _CB)], idx_cen)
        pltpu.sync_copy(neg_idx_hbm.at[pl.ds(cbase * _NEG, _CB * _NEG)],
                        idx_neg)
        copies = []
        for off in range(0, _CB * _CTX, 128):
            n = min(128, _CB * _CTX - off)
            copies.append(pltpu.async_copy(
                in_emb_hbm.at[idx_ctx.at[pl.ds(off, n)]],
                rows_ctx.at[pl.ds(off, n)], sem))
        copies.append(pltpu.async_copy(out_emb_hbm.at[idx_cen], rows_cen,
                                       sem))
        for off in range(0, _CB * _NEG, 128):
            n = min(128, _CB * _NEG - off)
            copies.append(pltpu.async_copy(
                out_emb_hbm.at[idx_neg.at[pl.ds(off, n)]],
                rows_neg.at[pl.ds(off, n)], sem))
        for cp in copies:
            cp.wait()

        lane = lax.broadcasted_iota(jnp.int32, (_L,), 0)
        perms = [(lane + sh) % _L for sh in (8, 4, 2, 1)]
        lane_masks = [lane == i for i in range(_L)]

        dnums = lax.GatherDimensionNumbers(
            offset_dims=(), collapsed_slice_dims=(0,), start_index_map=(0,))

        def hsum(vec):
            for p in perms:
                rot = lax.gather(
                    vec, p[:, None], dimension_numbers=dnums,
                    slice_sizes=(1,),
                    mode=lax.GatherScatterMode.PROMISE_IN_BOUNDS)
                vec = vec + rot
            return vec

        def row_body(c, carry2):
            s = []
            for q in range(_DIM // _L):
                a = rows_ctx[c * _CTX, pl.ds(q * _L, _L)]
                for j in range(1, _CTX):
                    a = a + rows_ctx[c * _CTX + j, pl.ds(q * _L, _L)]
                s.append(a * (1.0 / _CTX))
            row_off = c * _SCORE_COLS

            t = s[0] * rows_cen[c, pl.ds(0, _L)]
            for q in range(1, _DIM // _L):
                t = t + s[q] * rows_cen[c, pl.ds(q * _L, _L)]
            out_lo = hsum(t)
            out_hi = jnp.zeros((_L,), jnp.float32)
            for k in range(_NEG):
                u = s[0] * rows_neg[c * _NEG + k, pl.ds(0, _L)]
                for q in range(1, _DIM // _L):
                    u = u + s[q] * rows_neg[c * _NEG + k, pl.ds(q * _L, _L)]
                tot = hsum(u)
                col = 1 + k
                if col < _L:
                    out_lo = jnp.where(lane_masks[col], tot, out_lo)
                else:
                    out_hi = jnp.where(lane_masks[col - _L], tot, out_hi)
            scores_v[pl.ds(row_off, _L)] = out_lo
            scores_v[pl.ds(row_off + _L, _L)] = out_hi
            return carry2

        lax.fori_loop(0, _CB, row_body, 0)
        pltpu.sync_copy(
            scores_v,
            scores_hbm.at[pl.ds(cbase * _SCORE_COLS, _CB * _SCORE_COLS)])
        return carry

    lax.fori_loop(0, _NCH, chunk_body, 0)


def _loss_body(scores_ref, out_ref):
    s = scores_ref[...]
    col = lax.broadcasted_iota(jnp.int32, s.shape, 1)
    y = jnp.where(col == 0, s, -s)
    term = -jnp.log(jax.nn.sigmoid(y))
    term = jnp.where(col <= _NEG, term, 0.0)
    out_ref[...] = (jnp.sum(term) * (1.0 / _BATCH)).reshape(1, 1)


def kernel(context_words, center_word, negative_samples, in_embeddings,
           out_embeddings):
    ctx_flat = context_words.reshape(-1).astype(jnp.int32)
    cen = center_word.astype(jnp.int32)
    neg_flat = negative_samples.reshape(-1).astype(jnp.int32)

    mesh = plsc.VectorSubcoreMesh(core_axis_name="c", subcore_axis_name="s")
    scores = pl.kernel(
        _sc_scores_body,
        out_type=jax.ShapeDtypeStruct((_BATCH * _SCORE_COLS,), jnp.float32),
        mesh=mesh,
        scratch_types=[
            pltpu.VMEM((_CB * _CTX,), jnp.int32),
            pltpu.VMEM((_CB,), jnp.int32),
            pltpu.VMEM((_CB * _NEG,), jnp.int32),
            pltpu.VMEM((_CB * _CTX, _DIM), jnp.float32),
            pltpu.VMEM((_CB, _DIM), jnp.float32),
            pltpu.VMEM((_CB * _NEG, _DIM), jnp.float32),
            pltpu.VMEM((_CB * _SCORE_COLS,), jnp.float32),
            pltpu.SemaphoreType.DMA,
        ],
        compiler_params=pltpu.CompilerParams(use_tc_tiling_on_sc=False),
    )(ctx_flat, cen, neg_flat, in_embeddings, out_embeddings)

    loss2d = pl.pallas_call(
        _loss_body,
        out_shape=jax.ShapeDtypeStruct((1, 1), jnp.float32),
    )(scores.reshape(_BATCH, _SCORE_COLS))
    return loss2d[0, 0]
